# Initial kernel scaffold; baseline (speedup 1.0000x reference)
import jax
import jax.numpy as jnp
from jax.experimental import pallas as pl


def kernel(x_bs_c, pe):
    raise NotImplementedError("write your pallas kernel here")


# speedup vs baseline: 1.6016x; 1.6016x over previous
import functools

import jax
import jax.numpy as jnp
from jax import lax
from jax.experimental import pallas as pl
from jax.experimental.pallas import tpu as pltpu
from jax.experimental.pallas import tpu_sc as plsc


def _make_sc_broadcast(B, S, C, n_cores, n_subcores, chunk):
    n_workers = n_cores * n_subcores
    rows_per_w = S // n_workers
    n_chunks = rows_per_w // chunk
    mesh = plsc.VectorSubcoreMesh(core_axis_name="c", subcore_axis_name="s")

    @functools.partial(
        pl.kernel,
        mesh=mesh,
        out_type=jax.ShapeDtypeStruct((B, S, C), jnp.float32),
        scratch_types=[
            pltpu.VMEM((chunk, C), jnp.float32),
            pltpu.VMEM((chunk, C), jnp.float32),
            pltpu.SemaphoreType.DMA,
            pltpu.SemaphoreType.DMA,
            pltpu.SemaphoreType.DMA,
            pltpu.SemaphoreType.DMA,
        ],
    )
    def sc_broadcast(pe_hbm, out_hbm, buf0, buf1, lsem0, lsem1, ssem0, ssem1):
        wid = lax.axis_index("s") * n_cores + lax.axis_index("c")
        base = wid * rows_per_w

        bufs = (buf0, buf1)
        lsems = (lsem0, lsem1)
        ssems = (ssem0, ssem1)
        loads = [None, None]
        stores = [None, None]

        def start_load(k):
            j = k % 2
            loads[j] = pltpu.async_copy(
                pe_hbm.at[pl.ds(base + k * chunk, chunk)], bufs[j], lsems[j]
            )

        def start_stores(k):
            j = k % 2
            stores[j] = [
                pltpu.async_copy(
                    bufs[j],
                    out_hbm.at[b, pl.ds(base + k * chunk, chunk)],
                    ssems[j],
                )
                for b in range(B)
            ]

        start_load(0)
        for k in range(n_chunks):
            j = k % 2
            if k + 1 < n_chunks:
                if stores[(k + 1) % 2] is not None:
                    for cp in stores[(k + 1) % 2]:
                        cp.wait()
                    stores[(k + 1) % 2] = None
                start_load(k + 1)
            loads[j].wait()
            start_stores(k)
        for j in range(2):
            if stores[j] is not None:
                for cp in stores[j]:
                    cp.wait()

    return sc_broadcast


def kernel(x_bs_c, pe):
    B, S, C = x_bs_c.shape
    try:
        info = plsc.get_sparse_core_info()
        n_cores, n_subcores = info.num_cores, info.num_subcores
    except Exception:
        n_cores, n_subcores = 2, 16
    chunk = 32
    assert S % (n_cores * n_subcores * chunk) == 0
    return _make_sc_broadcast(B, S, C, n_cores, n_subcores, chunk)(pe)
